# baseline (speedup 1.0000x reference)
import functools

import jax
import jax.numpy as jnp
from jax import lax
from jax.experimental import pallas as pl
from jax.experimental.pallas import tpu as pltpu
from jax.experimental.pallas import tpu_sc as plsc

C = 128
E = 160000
NC, NS = 2, 16
CH = 80
NCHUNKS = 2048
E_PAD = NCHUNKS * CH
NPAIRS = NCHUNKS // 2
P0 = 1024
MAX_WCH = 2 * ((max(P0, NPAIRS - P0) + NS - 1) // NS)
EI = E_PAD + MAX_WCH * CH
ME = 640


def _sc_combine(table, idx_t):
    mesh = plsc.VectorSubcoreMesh(core_axis_name="c", subcore_axis_name="s")

    @functools.partial(
        pl.kernel,
        out_type=jax.ShapeDtypeStruct((E_PAD, 4 * C), jnp.float32),
        mesh=mesh,
        scratch_types=[
            [pltpu.VMEM((CH,), jnp.int32) for _ in range(4)],
            [pltpu.VMEM((CH,), jnp.int32) for _ in range(4)],
            [pltpu.VMEM((CH, C), jnp.float32) for _ in range(4)],
            [pltpu.VMEM((CH, C), jnp.float32) for _ in range(4)],
            pltpu.SemaphoreType.DMA,
            pltpu.SemaphoreType.DMA,
            pltpu.SemaphoreType.DMA,
            pltpu.SemaphoreType.DMA,
            pltpu.SemaphoreType.DMA,
            pltpu.SemaphoreType.DMA,
        ],
    )
    def k(table_ref, idx_ref, out_ref, idxa, idxb, seta, setb,
          isem0, isem1, gsem0, gsem1, ssem0, ssem1):
        cid = lax.axis_index("c")
        sid = lax.axis_index("s")
        idxs = (idxa, idxb)
        sets = (seta, setb)
        isems = (isem0, isem1)
        gsems = (gsem0, gsem1)
        ssems = (ssem0, ssem1)

        my_np = jnp.where(cid == 0, P0 // NS, (NPAIRS - P0) // NS)
        my_pair0 = jnp.where(cid == 0, 0, P0) + sid * my_np
        c0 = my_pair0 * 2
        start_edge = c0 * CH

        def idx_descs(kc, st):
            return [
                pltpu.make_async_copy(
                    idx_ref.at[pl.ds(j * EI + kc * CH, CH)],
                    idxs[st][j], isems[st])
                for j in range(4)
            ]

        def gather_descs(st):
            return [
                pltpu.make_async_copy(
                    table_ref.at[idxs[st][j]], sets[st][j], gsems[st])
                for j in range(4)
            ]

        def scatter_descs(kc, st):
            s = kc * CH
            return [
                pltpu.make_async_copy(
                    sets[st][j],
                    out_ref.at[pl.ds(s, CH), pl.ds(j * C, C)], ssems[st])
                for j in range(4)
            ]

        def compute(st):
            g0, g1, g2, g3 = sets[st]

            @pl.loop(0, CH)
            def edge(e):
                @pl.loop(0, C // 16)
                def chan(r):
                    sl = pl.ds(r * 16, 16)
                    v1 = g0[e, sl]
                    v2 = g1[e, sl]
                    v3 = g2[e, sl]
                    v4 = g3[e, sl]
                    g0[e, sl] = v1 + v3
                    g1[e, sl] = v2 + v4
                    g2[e, sl] = jnp.abs(v1 - v3)
                    g3[e, sl] = jnp.abs(v2 - v4)

        @pl.when(my_np > 0)
        def _run():
            for d in idx_descs(c0, 0):
                d.start()
            for d in idx_descs(c0, 0):
                d.wait()
            for d in gather_descs(0):
                d.start()
            for d in idx_descs(c0 + 1, 1):
                d.start()

            @pl.loop(0, my_np)
            def pair(i):
                kc = c0 + i * 2

                @pl.when(i >= 1)
                def _():
                    for d in scatter_descs(kc - 1, 1):
                        d.wait()

                for d in idx_descs(kc + 1, 1):
                    d.wait()
                for d in gather_descs(1):
                    d.start()
                for d in gather_descs(0):
                    d.wait()

                @pl.when(i + 1 < my_np)
                def _():
                    for d in idx_descs(kc + 2, 0):
                        d.start()

                compute(0)
                for d in scatter_descs(kc, 0):
                    d.start()

                for d in scatter_descs(kc, 0):
                    d.wait()

                @pl.when(i + 1 < my_np)
                def _():
                    for d in idx_descs(kc + 2, 0):
                        d.wait()
                    for d in gather_descs(0):
                        d.start()

                for d in gather_descs(1):
                    d.wait()

                @pl.when(i + 1 < my_np)
                def _():
                    for d in idx_descs(kc + 3, 1):
                        d.start()

                compute(1)
                for d in scatter_descs(kc + 1, 1):
                    d.start()

            for d in scatter_descs(c0 + my_np * 2 - 1, 1):
                d.wait()

    return k(table, idx_t)


def _tc_layer1(x0, combos, w0t, wrt, b_row):
    def body(x0_ref, cb_ref, w0_ref, wr_ref, b_ref, y_ref):
        acc = jnp.dot(x0_ref[...], w0_ref[...],
                      preferred_element_type=jnp.float32)
        acc = acc + jnp.dot(cb_ref[...], wr_ref[...],
                            preferred_element_type=jnp.float32)
        y_ref[...] = jnp.maximum(acc + b_ref[...], 0.0)

    return pl.pallas_call(
        body,
        grid=(E // ME,),
        in_specs=[
            pl.BlockSpec((ME, C), lambda i: (i, 0)),
            pl.BlockSpec((ME, 4 * C), lambda i: (i, 0)),
            pl.BlockSpec((C, C), lambda i: (0, 0)),
            pl.BlockSpec((4 * C, C), lambda i: (0, 0)),
            pl.BlockSpec((1, C), lambda i: (0, 0)),
        ],
        out_specs=pl.BlockSpec((ME, C), lambda i: (i, 0)),
        out_shape=jax.ShapeDtypeStruct((E, C), jnp.float32),
    )(x0, combos, w0t, wrt, b_row)


def _tc_layer2(y, combos, w0, wr, b_col):
    def body(y_ref, cb_ref, w0_ref, wr_ref, b_ref, o_ref):
        acc = lax.dot_general(w0_ref[...], y_ref[...],
                              (((1,), (1,)), ((), ())),
                              preferred_element_type=jnp.float32)
        acc = acc + lax.dot_general(wr_ref[...], cb_ref[...],
                                    (((1,), (1,)), ((), ())),
                                    preferred_element_type=jnp.float32)
        o_ref[...] = jnp.maximum(acc + b_ref[...], 0.0)[None]

    return pl.pallas_call(
        body,
        grid=(E // ME,),
        in_specs=[
            pl.BlockSpec((ME, C), lambda i: (i, 0)),
            pl.BlockSpec((ME, 4 * C), lambda i: (i, 0)),
            pl.BlockSpec((C, C), lambda i: (0, 0)),
            pl.BlockSpec((C, 4 * C), lambda i: (0, 0)),
            pl.BlockSpec((C, 1), lambda i: (0, 0)),
        ],
        out_specs=pl.BlockSpec((1, C, ME), lambda i: (0, 0, i)),
        out_shape=jax.ShapeDtypeStruct((1, C, E), jnp.float32),
    )(y, combos, w0, wr, b_col)


def kernel(fe, gemm_edges, W1, b1, W2, b2):
    x_t = fe[0, :, :, 0].T
    idx_pad = jnp.pad(gemm_edges, ((0, EI - E), (0, 0)))
    idx_t = idx_pad.T.reshape(-1)

    w0t1 = W1[:, :, 0].T
    wrt1 = W1[:, :, 1:].transpose(2, 1, 0).reshape(4 * C, C)
    w02 = W2[:, :, 0]
    wr2 = W2[:, :, 1:].transpose(0, 2, 1).reshape(C, 4 * C)

    combos1 = _sc_combine(x_t, idx_t)
    y1 = _tc_layer1(x_t, combos1, w0t1, wrt1, b1[None, :])
    combos2 = _sc_combine(y1, idx_t)
    return _tc_layer2(y1, combos2, w02, wr2, b2[:, None])


# speedup vs baseline: 1.3226x; 1.3226x over previous
import functools

import jax
import jax.numpy as jnp
from jax import lax
from jax.experimental import pallas as pl
from jax.experimental.pallas import tpu as pltpu
from jax.experimental.pallas import tpu_sc as plsc

C = 128
E = 160000
NC, NS = 2, 16
CH = 80
NCHUNKS = 2048
E_PAD = NCHUNKS * CH
NPAIRS = NCHUNKS // 2
P0 = 800
MAX_WCH = 2 * ((max(P0, NPAIRS - P0) + NS - 1) // NS)
EI = E_PAD + MAX_WCH * CH
ME = 1280


def _sc_combine(table, idx_t):
    mesh = plsc.VectorSubcoreMesh(core_axis_name="c", subcore_axis_name="s")

    @functools.partial(
        pl.kernel,
        out_type=jax.ShapeDtypeStruct((E_PAD, 4 * C), jnp.float32),
        mesh=mesh,
        scratch_types=[
            [pltpu.VMEM((CH,), jnp.int32) for _ in range(4)],
            [pltpu.VMEM((CH,), jnp.int32) for _ in range(4)],
            [pltpu.VMEM((CH, C), jnp.float32) for _ in range(4)],
            [pltpu.VMEM((CH, C), jnp.float32) for _ in range(4)],
            pltpu.SemaphoreType.DMA,
            pltpu.SemaphoreType.DMA,
            pltpu.SemaphoreType.DMA,
            pltpu.SemaphoreType.DMA,
            pltpu.SemaphoreType.DMA,
            pltpu.SemaphoreType.DMA,
        ],
    )
    def k(table_ref, idx_ref, out_ref, idxa, idxb, seta, setb,
          isem0, isem1, gsem0, gsem1, ssem0, ssem1):
        cid = lax.axis_index("c")
        sid = lax.axis_index("s")
        idxs = (idxa, idxb)
        sets = (seta, setb)
        isems = (isem0, isem1)
        gsems = (gsem0, gsem1)
        ssems = (ssem0, ssem1)

        my_np = jnp.where(cid == 0, P0 // NS, (NPAIRS - P0) // NS)
        my_pair0 = jnp.where(cid == 0, 0, P0) + sid * my_np
        c0 = my_pair0 * 2
        start_edge = c0 * CH

        def idx_descs(kc, st):
            return [
                pltpu.make_async_copy(
                    idx_ref.at[pl.ds(j * EI + kc * CH, CH)],
                    idxs[st][j], isems[st])
                for j in range(4)
            ]

        def gather_descs(st):
            return [
                pltpu.make_async_copy(
                    table_ref.at[idxs[st][j]], sets[st][j], gsems[st])
                for j in range(4)
            ]

        def scatter_descs(kc, st):
            s = kc * CH
            return [
                pltpu.make_async_copy(
                    sets[st][j],
                    out_ref.at[pl.ds(s, CH), pl.ds(j * C, C)], ssems[st])
                for j in range(4)
            ]

        def compute(st):
            g0, g1, g2, g3 = sets[st]

            @pl.loop(0, CH)
            def edge(e):
                @pl.loop(0, C // 16)
                def chan(r):
                    sl = pl.ds(r * 16, 16)
                    v1 = g0[e, sl]
                    v2 = g1[e, sl]
                    v3 = g2[e, sl]
                    v4 = g3[e, sl]
                    g0[e, sl] = v1 + v3
                    g1[e, sl] = v2 + v4
                    g2[e, sl] = jnp.abs(v1 - v3)
                    g3[e, sl] = jnp.abs(v2 - v4)

        @pl.when(my_np > 0)
        def _run():
            for d in idx_descs(c0, 0):
                d.start()
            for d in idx_descs(c0, 0):
                d.wait()
            for d in gather_descs(0):
                d.start()
            for d in idx_descs(c0 + 1, 1):
                d.start()

            @pl.loop(0, my_np)
            def pair(i):
                kc = c0 + i * 2

                @pl.when(i >= 1)
                def _():
                    for d in scatter_descs(kc - 1, 1):
                        d.wait()

                for d in idx_descs(kc + 1, 1):
                    d.wait()
                for d in gather_descs(1):
                    d.start()
                for d in gather_descs(0):
                    d.wait()

                @pl.when(i + 1 < my_np)
                def _():
                    for d in idx_descs(kc + 2, 0):
                        d.start()

                compute(0)
                for d in scatter_descs(kc, 0):
                    d.start()

                for d in scatter_descs(kc, 0):
                    d.wait()

                @pl.when(i + 1 < my_np)
                def _():
                    for d in idx_descs(kc + 2, 0):
                        d.wait()
                    for d in gather_descs(0):
                        d.start()

                for d in gather_descs(1):
                    d.wait()

                @pl.when(i + 1 < my_np)
                def _():
                    for d in idx_descs(kc + 3, 1):
                        d.start()

                compute(1)
                for d in scatter_descs(kc + 1, 1):
                    d.start()

            for d in scatter_descs(c0 + my_np * 2 - 1, 1):
                d.wait()

    return k(table, idx_t)


def _tc_layer1(x0, combos, w0t, wrt, b_row):
    def body(x0_ref, cb_ref, w0_ref, wr_ref, b_ref, y_ref):
        acc = jnp.dot(x0_ref[...], w0_ref[...],
                      preferred_element_type=jnp.float32)
        acc = acc + jnp.dot(cb_ref[...], wr_ref[...],
                            preferred_element_type=jnp.float32)
        y_ref[...] = jnp.maximum(acc + b_ref[...], 0.0)

    return pl.pallas_call(
        body,
        grid=(E // ME,),
        in_specs=[
            pl.BlockSpec((ME, C), lambda i: (i, 0)),
            pl.BlockSpec((ME, 4 * C), lambda i: (i, 0)),
            pl.BlockSpec((C, C), lambda i: (0, 0)),
            pl.BlockSpec((4 * C, C), lambda i: (0, 0)),
            pl.BlockSpec((1, C), lambda i: (0, 0)),
        ],
        out_specs=pl.BlockSpec((ME, C), lambda i: (i, 0)),
        out_shape=jax.ShapeDtypeStruct((E, C), jnp.float32),
    )(x0, combos, w0t, wrt, b_row)


def _tc_layer2(y, combos, w0, wr, b_col):
    def body(y_ref, cb_ref, w0_ref, wr_ref, b_ref, o_ref):
        acc = lax.dot_general(w0_ref[...], y_ref[...],
                              (((1,), (1,)), ((), ())),
                              preferred_element_type=jnp.float32)
        acc = acc + lax.dot_general(wr_ref[...], cb_ref[...],
                                    (((1,), (1,)), ((), ())),
                                    preferred_element_type=jnp.float32)
        o_ref[...] = jnp.maximum(acc + b_ref[...], 0.0)[None]

    return pl.pallas_call(
        body,
        grid=(E // ME,),
        in_specs=[
            pl.BlockSpec((ME, C), lambda i: (i, 0)),
            pl.BlockSpec((ME, 4 * C), lambda i: (i, 0)),
            pl.BlockSpec((C, C), lambda i: (0, 0)),
            pl.BlockSpec((C, 4 * C), lambda i: (0, 0)),
            pl.BlockSpec((C, 1), lambda i: (0, 0)),
        ],
        out_specs=pl.BlockSpec((1, C, ME), lambda i: (0, 0, i)),
        out_shape=jax.ShapeDtypeStruct((1, C, E), jnp.float32),
    )(y, combos, w0, wr, b_col)


def kernel(fe, gemm_edges, W1, b1, W2, b2):
    x_t = fe[0, :, :, 0].T
    idx_pad = jnp.pad(gemm_edges, ((0, EI - E), (0, 0)))
    idx_t = idx_pad.T.reshape(-1)

    w0t1 = W1[:, :, 0].T
    wrt1 = W1[:, :, 1:].transpose(2, 1, 0).reshape(4 * C, C)
    w02 = W2[:, :, 0]
    wr2 = W2[:, :, 1:].transpose(0, 2, 1).reshape(C, 4 * C)

    combos1 = _sc_combine(x_t, idx_t)
    y1 = _tc_layer1(x_t, combos1, w0t1, wrt1, b1[None, :])
    combos2 = _sc_combine(y1, idx_t)
    return _tc_layer2(y1, combos2, w02, wr2, b2[:, None])
